# baseline (speedup 1.0000x reference)
import functools

import jax
import jax.numpy as jnp
from jax import lax
from jax.experimental import pallas as pl
from jax.experimental.pallas import tpu as pltpu
from jax.experimental.pallas import tpu_sc as plsc

N_NODES = 10000
D = 128
NPAD = 10240
ROWS_PER_TILE = NPAD // 16
ZROWS = 64
NDEN = 10016
CHUNK = 64
GRP = 8
NTILES = 32


def _prologue_body(x_ref, w_ref, asv_ref, adv_ref,
                   h_ref, as_ref, ad_ref, cs_ref, cd_ref):
    h = jnp.dot(x_ref[...], w_ref[...], preferred_element_type=jnp.float32)
    h_ref[...] = h
    asb = jnp.dot(h, asv_ref[...], preferred_element_type=jnp.float32)
    adb = jnp.dot(h, adv_ref[...], preferred_element_type=jnp.float32)
    as_ref[...] = asb
    ad_ref[...] = adb
    ms = jnp.max(asb).reshape(1, 1)
    md = jnp.max(adb).reshape(1, 1)
    i = pl.program_id(0)

    @pl.when(i == 0)
    def _():
        cs_ref[...] = ms
        cd_ref[...] = md

    @pl.when(i > 0)
    def _():
        cs_ref[...] = jnp.maximum(cs_ref[...], ms)
        cd_ref[...] = jnp.maximum(cd_ref[...], md)


def _make_sc_kernel(nchunks):
    mesh = plsc.VectorSubcoreMesh(core_axis_name="c", subcore_axis_name="s")

    @functools.partial(
        pl.kernel,
        out_type=(
            jax.ShapeDtypeStruct((2 * NPAD, D), jnp.float32),
            jax.ShapeDtypeStruct((NTILES, NDEN), jnp.float32),
        ),
        mesh=mesh,
        compiler_params=pltpu.CompilerParams(needs_layout_passes=False),
        scratch_types=[
            pltpu.VMEM((NDEN,), jnp.float32),
            pltpu.VMEM((NDEN,), jnp.float32),
            pltpu.VMEM((NDEN,), jnp.float32),
            pltpu.VMEM((GRP, CHUNK), jnp.int32),
            pltpu.VMEM((GRP, CHUNK), jnp.int32),
            pltpu.VMEM((CHUNK,), jnp.float32),
            pltpu.VMEM((CHUNK, D), jnp.float32),
            pltpu.VMEM((CHUNK, D), jnp.float32),
            pltpu.VMEM((16,), jnp.float32),
            pltpu.VMEM_SHARED((NPAD, D), jnp.float32),
            pltpu.SemaphoreType.DMA,
            pltpu.SemaphoreType.DMA,
        ],
    )
    def sc_kernel(h_hbm, src_hbm, dst_hbm, asrc_hbm, adst_hbm, c_hbm,
                  acc_hbm, den_hbm,
                  asrc_v, adst_v, den_v, srcp, dstp, exbuf, rows0, rows1,
                  cvec, spmem_acc, gsem0, gsem1):
        c = lax.axis_index("c")
        s = lax.axis_index("s")
        wid = s * 2 + c

        pltpu.sync_copy(asrc_hbm, asrc_v)
        pltpu.sync_copy(adst_hbm, adst_v)
        pltpu.sync_copy(c_hbm, cvec)
        cval = cvec[...]

        zero16 = jnp.zeros((16,), jnp.float32)

        def _zero_rows(r, carry):
            for j in range(D // 16):
                rows0[r, pl.ds(j * 16, 16)] = zero16
            return carry
        lax.fori_loop(0, CHUNK, _zero_rows, 0)

        def _zero_den(i, carry):
            den_v[pl.ds(i * 16, 16)] = zero16
            return carry
        lax.fori_loop(0, NDEN // 16, _zero_den, 0)

        base = s * ROWS_PER_TILE
        def _zero_acc(k, carry):
            pltpu.sync_copy(rows0, spmem_acc.at[pl.ds(base + k * CHUNK, CHUNK)])
            return carry
        lax.fori_loop(0, ROWS_PER_TILE // CHUNK, _zero_acc, 0)

        plsc.subcore_barrier()

        rows = (rows0, rows1)
        gsems = (gsem0, gsem1)

        def _compute_ex(jg):
            def _exgroup(g, carry2):
                si = srcp[jg, pl.ds(g * 16, 16)]
                di = dstp[jg, pl.ds(g * 16, 16)]
                e = (plsc.load_gather(asrc_v, [si])
                     + plsc.load_gather(adst_v, [di]))
                e = jnp.where(e > 0, e, 0.2 * e)
                ex = jnp.exp(e - cval)
                exbuf[pl.ds(g * 16, 16)] = ex
                plsc.addupdate_scatter(den_v, [di], ex)
                return carry2
            lax.fori_loop(0, CHUNK // 16, _exgroup, 0)

        def _scale_rows(p):
            rv = rows[p]
            def _scale(r, carry2):
                a = plsc.load_gather(exbuf, [jnp.full((16,), r, jnp.int32)])
                for jj in range(D // 16):
                    rv[r, pl.ds(jj * 16, 16)] = rv[r, pl.ds(jj * 16, 16)] * a
                return carry2
            lax.fori_loop(0, CHUNK, _scale, 0)

        def _issue_gather(jg, p):
            pltpu.async_copy(h_hbm.at[srcp.at[jg]], rows[p], gsems[p])

        def _wait_gather(jg, p):
            pltpu.make_async_copy(
                h_hbm.at[srcp.at[jg]], rows[p], gsems[p]).wait()

        def _do_chunk(jg, p):
            _compute_ex(jg)
            _wait_gather(jg, p)
            _scale_rows(p)
            pltpu.sync_copy(rows[p], spmem_acc.at[dstp.at[jg]], add=True)

        npairs = nchunks // 2

        def _pair(i, carry):
            t0 = 2 * i
            g = t0 // GRP
            jg = t0 - g * GRP

            @pl.when(jg == 0)
            def _():
                pltpu.sync_copy(src_hbm.at[wid, g], srcp)
                pltpu.sync_copy(dst_hbm.at[wid, g], dstp)
                _issue_gather(0, 0)

            _issue_gather(jg + 1, 1)
            _do_chunk(jg, 0)

            @pl.when(jg < GRP - 2)
            def _():
                _issue_gather(jg + 2, 0)
            _do_chunk(jg + 1, 1)
            return carry
        lax.fori_loop(0, npairs, _pair, 0)

        plsc.subcore_barrier()

        pltpu.sync_copy(spmem_acc.at[pl.ds(base, ROWS_PER_TILE)],
                        acc_hbm.at[pl.ds(c * NPAD + base, ROWS_PER_TILE)])
        pltpu.sync_copy(den_v, den_hbm.at[wid])

    return sc_kernel


def _epilogue_body(acc0_ref, acc1_ref, den_ref, bias_ref, out_ref):
    a = acc0_ref[...] + acc1_ref[...]
    dsum = jnp.sum(den_ref[...], axis=0)
    y = a / (dsum[:, None] + 1e-16) + bias_ref[...]
    out_ref[...] = jnp.where(y > 0, y, jnp.exp(jnp.minimum(y, 0.0)) - 1.0)


def kernel(x, edge_index_l1, W, att_src, att_dst, bias):
    n = x.shape[0]
    e = edge_index_l1.shape[1]
    ntotal = e + n
    gsz = GRP * CHUNK
    ept = -(-ntotal // (NTILES * gsz)) * gsz
    epad = ept * NTILES
    nchunks = ept // CHUNK

    self_loops = jnp.arange(n, dtype=edge_index_l1.dtype)
    src = jnp.concatenate([edge_index_l1[0], self_loops])
    dst = jnp.concatenate([edge_index_l1[1], self_loops])
    pad_n = epad - ntotal
    src = jnp.pad(src, (0, pad_n), constant_values=n)
    dst = jnp.pad(dst, (0, pad_n), constant_values=n)
    src3 = src.reshape(NTILES, nchunks // GRP, GRP, CHUNK)
    dst3 = dst.reshape(NTILES, nchunks // GRP, GRP, CHUNK)
    x_pad = jnp.pad(x, ((0, NPAD - n), (0, 0)))

    BLK = 1024
    grid = NPAD // BLK
    h, a_src2, a_dst2, cs, cd = pl.pallas_call(
        _prologue_body,
        grid=(grid,),
        in_specs=[
            pl.BlockSpec((BLK, D), lambda i: (i, 0)),
            pl.BlockSpec((D, D), lambda i: (0, 0)),
            pl.BlockSpec((D, 1), lambda i: (0, 0)),
            pl.BlockSpec((D, 1), lambda i: (0, 0)),
        ],
        out_specs=[
            pl.BlockSpec((BLK, D), lambda i: (i, 0)),
            pl.BlockSpec((BLK, 1), lambda i: (i, 0)),
            pl.BlockSpec((BLK, 1), lambda i: (i, 0)),
            pl.BlockSpec((1, 1), lambda i: (0, 0)),
            pl.BlockSpec((1, 1), lambda i: (0, 0)),
        ],
        out_shape=[
            jax.ShapeDtypeStruct((NPAD, D), jnp.float32),
            jax.ShapeDtypeStruct((NPAD, 1), jnp.float32),
            jax.ShapeDtypeStruct((NPAD, 1), jnp.float32),
            jax.ShapeDtypeStruct((1, 1), jnp.float32),
            jax.ShapeDtypeStruct((1, 1), jnp.float32),
        ],
    )(x_pad, W, att_src.reshape(D, 1), att_dst.reshape(D, 1))

    csum = cs[0, 0] + cd[0, 0]
    cmax = jnp.where(csum > 0, csum, 0.2 * csum)
    cvec = jnp.full((16,), cmax, jnp.float32)

    acc, den = _make_sc_kernel(nchunks)(
        h, src3, dst3, a_src2.reshape(NPAD)[:NDEN],
        a_dst2.reshape(NPAD)[:NDEN], cvec)
    den = jnp.pad(den, ((0, 0), (0, NPAD - NDEN)))

    out = pl.pallas_call(
        _epilogue_body,
        grid=(grid,),
        in_specs=[
            pl.BlockSpec((BLK, D), lambda i: (i, 0)),
            pl.BlockSpec((BLK, D), lambda i: (i + NPAD // BLK, 0)),
            pl.BlockSpec((NTILES, BLK), lambda i: (0, i)),
            pl.BlockSpec((1, D), lambda i: (0, 0)),
        ],
        out_specs=pl.BlockSpec((BLK, D), lambda i: (i, 0)),
        out_shape=jax.ShapeDtypeStruct((NPAD, D), jnp.float32),
    )(acc, acc, den, bias.reshape(1, D))

    return out[:n]


# speedup vs baseline: 2.0437x; 2.0437x over previous
import functools

import jax
import jax.numpy as jnp
from jax import lax
from jax.experimental import pallas as pl
from jax.experimental.pallas import tpu as pltpu
from jax.experimental.pallas import tpu_sc as plsc

N_NODES = 10000
D = 128
NPAD = 10240
ROWS_PER_TILE = NPAD // 16
ZROWS = 64
NDEN = 10016
CHUNK = 64
GRP = 8
NTILES = 32


def _prologue_body(x_ref, w_ref, asv_ref, adv_ref,
                   h_ref, as_ref, ad_ref, cs_ref, cd_ref):
    h = jnp.dot(x_ref[...], w_ref[...], preferred_element_type=jnp.float32)
    h_ref[...] = h
    asb = jnp.dot(h, asv_ref[...], preferred_element_type=jnp.float32)
    adb = jnp.dot(h, adv_ref[...], preferred_element_type=jnp.float32)
    as_ref[...] = asb
    ad_ref[...] = adb
    ms = jnp.max(asb).reshape(1, 1)
    md = jnp.max(adb).reshape(1, 1)
    i = pl.program_id(0)

    @pl.when(i == 0)
    def _():
        cs_ref[...] = ms
        cd_ref[...] = md

    @pl.when(i > 0)
    def _():
        cs_ref[...] = jnp.maximum(cs_ref[...], ms)
        cd_ref[...] = jnp.maximum(cd_ref[...], md)


def _make_sc_kernel(nchunks):
    mesh = plsc.VectorSubcoreMesh(core_axis_name="c", subcore_axis_name="s")

    @functools.partial(
        pl.kernel,
        out_type=(
            jax.ShapeDtypeStruct((2 * NPAD, D), jnp.float32),
            jax.ShapeDtypeStruct((NTILES, NDEN), jnp.float32),
        ),
        mesh=mesh,
        compiler_params=pltpu.CompilerParams(needs_layout_passes=False),
        scratch_types=[
            pltpu.VMEM((NDEN,), jnp.float32),
            pltpu.VMEM((NDEN,), jnp.float32),
            pltpu.VMEM((NDEN,), jnp.float32),
            pltpu.VMEM((2, CHUNK), jnp.int32),
            pltpu.VMEM((CHUNK,), jnp.int32),
            pltpu.VMEM((CHUNK,), jnp.int32),
            pltpu.VMEM((CHUNK,), jnp.int32),
            pltpu.VMEM((CHUNK,), jnp.int32),
            pltpu.VMEM((CHUNK,), jnp.float32),
            pltpu.VMEM((CHUNK, D), jnp.float32),
            pltpu.VMEM((CHUNK, D), jnp.float32),
            pltpu.VMEM((16,), jnp.float32),
            pltpu.VMEM_SHARED((NPAD, D), jnp.float32),
            pltpu.SemaphoreType.DMA,
            pltpu.SemaphoreType.DMA,
        ],
    )
    def sc_kernel(h_hbm, eidx_hbm, asrc_hbm, adst_hbm, c_hbm,
                  acc_hbm, den_hbm,
                  asrc_v, adst_v, den_v, pk, srcb0, srcb1, dstb0, dstb1,
                  exbuf, rows0, rows1, cvec, spmem_acc, gsem0, gsem1):
        c = lax.axis_index("c")
        s = lax.axis_index("s")
        wid = s * 2 + c

        pltpu.sync_copy(asrc_hbm, asrc_v)
        pltpu.sync_copy(adst_hbm, adst_v)
        pltpu.sync_copy(c_hbm, cvec)
        cval = cvec[...]

        zero16 = jnp.zeros((16,), jnp.float32)

        def _zero_rows(r, carry):
            for j in range(D // 16):
                rows0[r, pl.ds(j * 16, 16)] = zero16
            return carry
        lax.fori_loop(0, CHUNK, _zero_rows, 0)

        def _zero_den(i, carry):
            den_v[pl.ds(i * 16, 16)] = zero16
            return carry
        lax.fori_loop(0, NDEN // 16, _zero_den, 0)

        base = s * ROWS_PER_TILE
        def _zero_acc(k, carry):
            pltpu.sync_copy(rows0, spmem_acc.at[pl.ds(base + k * CHUNK, CHUNK)])
            return carry
        lax.fori_loop(0, ROWS_PER_TILE // CHUNK, _zero_acc, 0)

        plsc.subcore_barrier()

        rows = (rows0, rows1)
        gsems = (gsem0, gsem1)
        srcbs = (srcb0, srcb1)
        dstbs = (dstb0, dstb1)

        def _compute_ex(p):
            def _exgroup(g, carry2):
                si = srcbs[p][pl.ds(g * 16, 16)]
                di = dstbs[p][pl.ds(g * 16, 16)]
                e = (plsc.load_gather(asrc_v, [si])
                     + plsc.load_gather(adst_v, [di]))
                e = jnp.where(e > 0, e, 0.2 * e)
                ex = jnp.exp(e - cval)
                exbuf[pl.ds(g * 16, 16)] = ex
                plsc.addupdate_scatter(den_v, [di], ex)
                return carry2
            lax.fori_loop(0, CHUNK // 16, _exgroup, 0)

        def _scale_rows(p):
            rv = rows[p]
            def _scale(r, carry2):
                a = plsc.load_gather(exbuf, [jnp.full((16,), r, jnp.int32)])
                for jj in range(D // 16):
                    rv[r, pl.ds(jj * 16, 16)] = rv[r, pl.ds(jj * 16, 16)] * a
                return carry2
            lax.fori_loop(0, CHUNK, _scale, 0)

        def _stage_and_gather(tnext, p):
            pltpu.sync_copy(eidx_hbm.at[wid, tnext], pk.at[p])
            sb, db = srcbs[p], dstbs[p]
            def _unpack(g, carry2):
                w = pk[p, pl.ds(g * 16, 16)]
                sb[pl.ds(g * 16, 16)] = lax.shift_right_logical(w, 14)
                db[pl.ds(g * 16, 16)] = lax.bitwise_and(
                    w, jnp.full((16,), 16383, jnp.int32))
                return carry2
            lax.fori_loop(0, CHUNK // 16, _unpack, 0)
            pltpu.async_copy(h_hbm.at[sb], rows[p], gsems[p])

        def _wait_gather(p):
            pltpu.make_async_copy(
                h_hbm.at[srcbs[p]], rows[p], gsems[p]).wait()

        def _do_chunk(t, p):
            _compute_ex(p)
            _wait_gather(p)
            _scale_rows(p)
            pltpu.sync_copy(rows[p], spmem_acc.at[dstbs[p]], add=True)

        npairs = nchunks // 2
        _stage_and_gather(0, 0)

        def _pair(i, carry):
            t0 = 2 * i
            _stage_and_gather(t0 + 1, 1)
            _do_chunk(t0, 0)

            @pl.when(i < npairs - 1)
            def _():
                _stage_and_gather(t0 + 2, 0)
            _do_chunk(t0 + 1, 1)
            return carry
        lax.fori_loop(0, npairs, _pair, 0)

        plsc.subcore_barrier()

        pltpu.sync_copy(spmem_acc.at[pl.ds(base, ROWS_PER_TILE)],
                        acc_hbm.at[pl.ds(c * NPAD + base, ROWS_PER_TILE)])
        pltpu.sync_copy(den_v, den_hbm.at[wid])

    return sc_kernel


def _epilogue_body(acc0_ref, acc1_ref, den_ref, bias_ref, out_ref):
    a = acc0_ref[...] + acc1_ref[...]
    dsum = jnp.sum(den_ref[...], axis=0)
    y = a / (dsum[:, None] + 1e-16) + bias_ref[...]
    out_ref[...] = jnp.where(y > 0, y, jnp.exp(jnp.minimum(y, 0.0)) - 1.0)


def kernel(x, edge_index_l1, W, att_src, att_dst, bias):
    n = x.shape[0]
    e = edge_index_l1.shape[1]
    ntotal = e + n
    ept = -(-ntotal // (NTILES * 2 * CHUNK)) * (2 * CHUNK)
    epad = ept * NTILES
    nchunks = ept // CHUNK

    self_loops = jnp.arange(n, dtype=edge_index_l1.dtype)
    src = jnp.concatenate([edge_index_l1[0], self_loops])
    dst = jnp.concatenate([edge_index_l1[1], self_loops])
    pad_n = epad - ntotal
    src = jnp.pad(src, (0, pad_n), constant_values=n)
    dst = jnp.pad(dst, (0, pad_n), constant_values=n)
    packed = src * jnp.int32(16384) + dst
    eidx3 = packed.reshape(NTILES, nchunks, CHUNK)
    x_pad = jnp.pad(x, ((0, NPAD - n), (0, 0)))

    BLK = 1024
    grid = NPAD // BLK
    h, a_src2, a_dst2, cs, cd = pl.pallas_call(
        _prologue_body,
        grid=(grid,),
        in_specs=[
            pl.BlockSpec((BLK, D), lambda i: (i, 0)),
            pl.BlockSpec((D, D), lambda i: (0, 0)),
            pl.BlockSpec((D, 1), lambda i: (0, 0)),
            pl.BlockSpec((D, 1), lambda i: (0, 0)),
        ],
        out_specs=[
            pl.BlockSpec((BLK, D), lambda i: (i, 0)),
            pl.BlockSpec((BLK, 1), lambda i: (i, 0)),
            pl.BlockSpec((BLK, 1), lambda i: (i, 0)),
            pl.BlockSpec((1, 1), lambda i: (0, 0)),
            pl.BlockSpec((1, 1), lambda i: (0, 0)),
        ],
        out_shape=[
            jax.ShapeDtypeStruct((NPAD, D), jnp.float32),
            jax.ShapeDtypeStruct((NPAD, 1), jnp.float32),
            jax.ShapeDtypeStruct((NPAD, 1), jnp.float32),
            jax.ShapeDtypeStruct((1, 1), jnp.float32),
            jax.ShapeDtypeStruct((1, 1), jnp.float32),
        ],
    )(x_pad, W, att_src.reshape(D, 1), att_dst.reshape(D, 1))

    csum = cs[0, 0] + cd[0, 0]
    cmax = jnp.where(csum > 0, csum, 0.2 * csum)
    cvec = jnp.full((16,), cmax, jnp.float32)

    acc, den = _make_sc_kernel(nchunks)(
        h, eidx3, a_src2.reshape(NPAD)[:NDEN],
        a_dst2.reshape(NPAD)[:NDEN], cvec)
    den = jnp.pad(den, ((0, 0), (0, NPAD - NDEN)))

    out = pl.pallas_call(
        _epilogue_body,
        grid=(grid,),
        in_specs=[
            pl.BlockSpec((BLK, D), lambda i: (i, 0)),
            pl.BlockSpec((BLK, D), lambda i: (i + NPAD // BLK, 0)),
            pl.BlockSpec((NTILES, BLK), lambda i: (0, i)),
            pl.BlockSpec((1, D), lambda i: (0, 0)),
        ],
        out_specs=pl.BlockSpec((BLK, D), lambda i: (i, 0)),
        out_shape=jax.ShapeDtypeStruct((NPAD, D), jnp.float32),
    )(acc, acc, den, bias.reshape(1, D))

    return out[:n]
